# baseline (speedup 1.0000x reference)
import functools

import jax
import jax.numpy as jnp
from jax import lax
from jax.experimental import pallas as pl
from jax.experimental.pallas import tpu as pltpu
from jax.experimental.pallas import tpu_sc as plsc

N = 10000
E = 320000
D = 128
H = 128
G = 16

NC = 1
NS = 16
NW = NC * NS
C = 128
K = 160
IB = 16
NB = 2
EW = K * C
EP = NW * EW
NP = 10240
BR = 2000


def _sc_agg(h, src2, dst2):
  mesh = plsc.VectorSubcoreMesh(core_axis_name="c", subcore_axis_name="s",
                                num_cores=NC)

  @functools.partial(
      pl.kernel,
      out_type=jax.ShapeDtypeStruct((NC, NP, D), jnp.float32),
      mesh=mesh,
      scratch_types=[
          pltpu.VMEM((IB, C), jnp.int32),
          pltpu.VMEM((IB, C), jnp.int32),
          pltpu.VMEM((NB, C, D), jnp.float32),
          pltpu.VMEM_SHARED((NP, D), jnp.float32),
          pltpu.SemaphoreType.DMA((NB,)),
          pltpu.SemaphoreType.DMA((NB,)),
          pltpu.SemaphoreType.DMA,
      ],
  )
  def k(h_hbm, src_hbm, dst_hbm, out_hbm, src_v, dst_v, rows_v, agg,
        gsem, ssem, xsem):
    c = lax.axis_index("c")
    s = lax.axis_index("s")
    chunk0 = (c * NS + s) * K

    def zrow(i, carry):
      def zlane(j, carry2):
        rows_v[0, i, pl.ds(j * 16, 16)] = jnp.zeros((16,), jnp.float32)
        return carry2
      return lax.fori_loop(0, D // 16, zlane, carry)
    lax.fori_loop(0, C, zrow, 0)
    nz = NP // NS // C
    for t in range(nz):
      pltpu.async_copy(rows_v.at[0],
                       agg.at[pl.ds(s * (NP // NS) + t * C, C)], xsem)
    for t in range(nz):
      pltpu.make_async_copy(rows_v.at[0], agg.at[pl.ds(t * C, C)],
                            xsem).wait()
    plsc.subcore_barrier()

    def g_start(j, b):
      pltpu.async_copy(h_hbm.at[src_v.at[j]], rows_v.at[b], gsem.at[b])

    def g_wait(j, b):
      pltpu.make_async_copy(h_hbm.at[src_v.at[j]], rows_v.at[b],
                            gsem.at[b]).wait()

    def s_wait(j, b):
      pltpu.make_async_copy(rows_v.at[b], agg.at[dst_v.at[j]],
                            ssem.at[b]).wait()

    def block(bi, carry):
      base = chunk0 + bi * IB
      pltpu.async_copy(src_hbm.at[pl.ds(base, IB)], src_v, xsem)
      pltpu.async_copy(dst_hbm.at[pl.ds(base, IB)], dst_v, xsem)
      pltpu.make_async_copy(src_hbm.at[pl.ds(0, IB)], src_v, xsem).wait()
      pltpu.make_async_copy(dst_hbm.at[pl.ds(0, IB)], dst_v, xsem).wait()
      g_start(0, 0)

      def step(i, carry2):
        for b in range(NB):
          jl = i * NB + b
          g_wait(jl, b)
          pltpu.async_copy(rows_v.at[b], agg.at[dst_v.at[jl]], ssem.at[b],
                           add=True)

          @pl.when(jl >= 1)
          def _():
            s_wait(jl - 1, (b - 1) % NB)

          @pl.when(jl + 1 < IB)
          def _():
            g_start(jl + 1, (b + 1) % NB)
        return carry2
      lax.fori_loop(0, IB // NB, step, 0)
      s_wait(IB - 1, (IB - 1) % NB)
      return carry
    lax.fori_loop(0, K // IB, block, 0)
    plsc.subcore_barrier()

    pltpu.sync_copy(agg.at[pl.ds(s * (NP // NS), NP // NS)],
                    out_hbm.at[c, pl.ds(s * (NP // NS), NP // NS)])

  return k(h, src2, dst2)


def _mlp_compute(h_ref, p_ref, w1_ref, b1_ref, w2_ref, b2_ref):
  m = h_ref[...] + p_ref[0]
  for q in range(1, NC):
    m = m + p_ref[q]
  m = jnp.dot(m, w1_ref[...], preferred_element_type=jnp.float32) + b1_ref[...]
  m = jnp.maximum(m, 0.0)
  m = jnp.dot(m, w2_ref[...], preferred_element_type=jnp.float32) + b2_ref[...]
  return m


def _mlp(h, p, w1, b1, w2, b2, relu_out):
  def body(h_ref, p_ref, w1_ref, b1_ref, w2_ref, b2_ref, o_ref):
    m = _mlp_compute(h_ref, p_ref, w1_ref, b1_ref, w2_ref, b2_ref)
    if relu_out:
      m = jnp.maximum(m, 0.0)
    o_ref[...] = m

  return pl.pallas_call(
      body,
      grid=(N // BR,),
      in_specs=[
          pl.BlockSpec((BR, D), lambda i: (i, 0)),
          pl.BlockSpec((NC, BR, D), lambda i: (0, i, 0)),
          pl.BlockSpec((D, H), lambda i: (0, 0)),
          pl.BlockSpec((1, H), lambda i: (0, 0)),
          pl.BlockSpec((H, H), lambda i: (0, 0)),
          pl.BlockSpec((1, H), lambda i: (0, 0)),
      ],
      out_specs=pl.BlockSpec((BR, H), lambda i: (i, 0)),
      out_shape=jax.ShapeDtypeStruct((N, H), jnp.float32),
  )(h, p, w1, b1.reshape(1, H), w2, b2.reshape(1, H))


def _mlp_pool(h, p, w1, b1, w2, b2, onehot):
  def body(h_ref, p_ref, w1_ref, b1_ref, w2_ref, b2_ref, oh_ref, o_ref):
    i = pl.program_id(0)
    m = _mlp_compute(h_ref, p_ref, w1_ref, b1_ref, w2_ref, b2_ref)
    rows = []
    for g in range(G):
      col = oh_ref[:, pl.ds(g, 1)]
      masked = jnp.where(col > 0.5, m, -jnp.inf)
      rows.append(jnp.max(masked, axis=0, keepdims=True))
    cur = jnp.concatenate(rows, axis=0)

    @pl.when(i == 0)
    def _():
      o_ref[...] = jnp.full((G, H), -jnp.inf, jnp.float32)
    o_ref[...] = jnp.maximum(o_ref[...], cur)

  return pl.pallas_call(
      body,
      grid=(N // BR,),
      in_specs=[
          pl.BlockSpec((BR, D), lambda i: (i, 0)),
          pl.BlockSpec((NC, BR, D), lambda i: (0, i, 0)),
          pl.BlockSpec((D, H), lambda i: (0, 0)),
          pl.BlockSpec((1, H), lambda i: (0, 0)),
          pl.BlockSpec((H, H), lambda i: (0, 0)),
          pl.BlockSpec((1, H), lambda i: (0, 0)),
          pl.BlockSpec((BR, G), lambda i: (i, 0)),
      ],
      out_specs=pl.BlockSpec((G, H), lambda i: (0, 0)),
      out_shape=jax.ShapeDtypeStruct((G, H), jnp.float32),
  )(h, p, w1, b1.reshape(1, H), w2, b2.reshape(1, H), onehot)


def kernel(x, edge_index, batch, W1_0, b1_0, W2_0, b2_0, W1_1, b1_1, W2_1,
           b2_1, W1_2, b1_2, W2_2, b2_2):
  params = [(W1_0, b1_0, W2_0, b2_0), (W1_1, b1_1, W2_1, b2_1),
            (W1_2, b1_2, W2_2, b2_2)]
  src = edge_index[0]
  dst = edge_index[1]
  srcp = jnp.concatenate([src, jnp.zeros((EP - E,), jnp.int32)])
  dstp = jnp.concatenate([dst, jnp.full((EP - E,), N, jnp.int32)])
  src2 = srcp.reshape(NW * K, C)
  dst2 = dstp.reshape(NW * K, C)
  onehot = (batch[:, None] == jnp.arange(G, dtype=jnp.int32)[None, :]
            ).astype(jnp.float32)

  h = x
  out = None
  for i, (w1, b1, w2, b2) in enumerate(params):
    p = _sc_agg(h, src2, dst2)
    if i < 2:
      h = _mlp(h, p, w1, b1, w2, b2, relu_out=True)
    else:
      out = _mlp_pool(h, p, w1, b1, w2, b2, onehot)
  return out


# speedup vs baseline: 1.2499x; 1.2499x over previous
import functools

import jax
import jax.numpy as jnp
from jax import lax
from jax.experimental import pallas as pl
from jax.experimental.pallas import tpu as pltpu
from jax.experimental.pallas import tpu_sc as plsc

N = 10000
E = 320000
D = 128
H = 128
G = 16

NC = 2
NS = 16
NW = NC * NS
K0 = 96
K1 = 64
C = 128
IB = 16
NB = 2
EP = NS * (K0 + K1) * C
NP = 10240
BR = 2000


def _sc_agg(h, src2, dst2):
  mesh = plsc.VectorSubcoreMesh(core_axis_name="c", subcore_axis_name="s",
                                num_cores=NC)

  @functools.partial(
      pl.kernel,
      out_type=jax.ShapeDtypeStruct((NC, NP, D), jnp.float32),
      mesh=mesh,
      scratch_types=[
          pltpu.VMEM((IB, C), jnp.int32),
          pltpu.VMEM((IB, C), jnp.int32),
          pltpu.VMEM((NB, C, D), jnp.float32),
          pltpu.VMEM_SHARED((NP, D), jnp.float32),
          pltpu.SemaphoreType.DMA((NB,)),
          pltpu.SemaphoreType.DMA((NB,)),
          pltpu.SemaphoreType.DMA,
      ],
  )
  def k(h_hbm, src_hbm, dst_hbm, out_hbm, src_v, dst_v, rows_v, agg,
        gsem, ssem, xsem):
    c = lax.axis_index("c")
    s = lax.axis_index("s")
    myk = jnp.where(c == 0, K0, K1)
    chunk0 = jnp.where(c == 0, s * K0, NS * K0 + s * K1)

    def zrow(i, carry):
      def zlane(j, carry2):
        rows_v[0, i, pl.ds(j * 16, 16)] = jnp.zeros((16,), jnp.float32)
        return carry2
      return lax.fori_loop(0, D // 16, zlane, carry)
    lax.fori_loop(0, C, zrow, 0)
    nz = NP // NS // C
    for t in range(nz):
      pltpu.async_copy(rows_v.at[0],
                       agg.at[pl.ds(s * (NP // NS) + t * C, C)], xsem)
    for t in range(nz):
      pltpu.make_async_copy(rows_v.at[0], agg.at[pl.ds(t * C, C)],
                            xsem).wait()
    plsc.subcore_barrier()

    def g_start(j, b):
      pltpu.async_copy(h_hbm.at[src_v.at[j]], rows_v.at[b], gsem.at[b])

    def g_wait(j, b):
      pltpu.make_async_copy(h_hbm.at[src_v.at[j]], rows_v.at[b],
                            gsem.at[b]).wait()

    def s_wait(j, b):
      pltpu.make_async_copy(rows_v.at[b], agg.at[dst_v.at[j]],
                            ssem.at[b]).wait()

    def block(bi, carry):
      base = chunk0 + bi * IB
      pltpu.async_copy(src_hbm.at[pl.ds(base, IB)], src_v, xsem)
      pltpu.async_copy(dst_hbm.at[pl.ds(base, IB)], dst_v, xsem)
      pltpu.make_async_copy(src_hbm.at[pl.ds(0, IB)], src_v, xsem).wait()
      pltpu.make_async_copy(dst_hbm.at[pl.ds(0, IB)], dst_v, xsem).wait()
      g_start(0, 0)

      def step(i, carry2):
        for b in range(NB):
          jl = i * NB + b
          g_wait(jl, b)
          pltpu.async_copy(rows_v.at[b], agg.at[dst_v.at[jl]], ssem.at[b],
                           add=True)

          @pl.when(jl >= 1)
          def _():
            s_wait(jl - 1, (b - 1) % NB)

          @pl.when(jl + 1 < IB)
          def _():
            g_start(jl + 1, (b + 1) % NB)
        return carry2
      lax.fori_loop(0, IB // NB, step, 0)
      s_wait(IB - 1, (IB - 1) % NB)
      return carry
    lax.fori_loop(0, myk // IB, block, 0)
    plsc.subcore_barrier()

    pltpu.sync_copy(agg.at[pl.ds(s * (NP // NS), NP // NS)],
                    out_hbm.at[c, pl.ds(s * (NP // NS), NP // NS)])

  return k(h, src2, dst2)


def _mlp_compute(h_ref, p_ref, w1_ref, b1_ref, w2_ref, b2_ref):
  m = h_ref[...] + p_ref[0]
  for q in range(1, NC):
    m = m + p_ref[q]
  m = jnp.dot(m, w1_ref[...], preferred_element_type=jnp.float32) + b1_ref[...]
  m = jnp.maximum(m, 0.0)
  m = jnp.dot(m, w2_ref[...], preferred_element_type=jnp.float32) + b2_ref[...]
  return m


def _mlp(h, p, w1, b1, w2, b2, relu_out):
  def body(h_ref, p_ref, w1_ref, b1_ref, w2_ref, b2_ref, o_ref):
    m = _mlp_compute(h_ref, p_ref, w1_ref, b1_ref, w2_ref, b2_ref)
    if relu_out:
      m = jnp.maximum(m, 0.0)
    o_ref[...] = m

  return pl.pallas_call(
      body,
      grid=(N // BR,),
      in_specs=[
          pl.BlockSpec((BR, D), lambda i: (i, 0)),
          pl.BlockSpec((NC, BR, D), lambda i: (0, i, 0)),
          pl.BlockSpec((D, H), lambda i: (0, 0)),
          pl.BlockSpec((1, H), lambda i: (0, 0)),
          pl.BlockSpec((H, H), lambda i: (0, 0)),
          pl.BlockSpec((1, H), lambda i: (0, 0)),
      ],
      out_specs=pl.BlockSpec((BR, H), lambda i: (i, 0)),
      out_shape=jax.ShapeDtypeStruct((N, H), jnp.float32),
  )(h, p, w1, b1.reshape(1, H), w2, b2.reshape(1, H))


def _mlp_pool(h, p, w1, b1, w2, b2, onehot):
  def body(h_ref, p_ref, w1_ref, b1_ref, w2_ref, b2_ref, oh_ref, o_ref):
    i = pl.program_id(0)
    m = _mlp_compute(h_ref, p_ref, w1_ref, b1_ref, w2_ref, b2_ref)
    rows = []
    for g in range(G):
      col = oh_ref[:, pl.ds(g, 1)]
      masked = jnp.where(col > 0.5, m, -jnp.inf)
      rows.append(jnp.max(masked, axis=0, keepdims=True))
    cur = jnp.concatenate(rows, axis=0)

    @pl.when(i == 0)
    def _():
      o_ref[...] = jnp.full((G, H), -jnp.inf, jnp.float32)
    o_ref[...] = jnp.maximum(o_ref[...], cur)

  return pl.pallas_call(
      body,
      grid=(N // BR,),
      in_specs=[
          pl.BlockSpec((BR, D), lambda i: (i, 0)),
          pl.BlockSpec((NC, BR, D), lambda i: (0, i, 0)),
          pl.BlockSpec((D, H), lambda i: (0, 0)),
          pl.BlockSpec((1, H), lambda i: (0, 0)),
          pl.BlockSpec((H, H), lambda i: (0, 0)),
          pl.BlockSpec((1, H), lambda i: (0, 0)),
          pl.BlockSpec((BR, G), lambda i: (i, 0)),
      ],
      out_specs=pl.BlockSpec((G, H), lambda i: (0, 0)),
      out_shape=jax.ShapeDtypeStruct((G, H), jnp.float32),
  )(h, p, w1, b1.reshape(1, H), w2, b2.reshape(1, H), onehot)


def kernel(x, edge_index, batch, W1_0, b1_0, W2_0, b2_0, W1_1, b1_1, W2_1,
           b2_1, W1_2, b1_2, W2_2, b2_2):
  params = [(W1_0, b1_0, W2_0, b2_0), (W1_1, b1_1, W2_1, b2_1),
            (W1_2, b1_2, W2_2, b2_2)]
  src = edge_index[0]
  dst = edge_index[1]
  srcp = jnp.concatenate([src, jnp.zeros((EP - E,), jnp.int32)])
  dstp = jnp.concatenate([dst, jnp.full((EP - E,), N, jnp.int32)])
  src2 = srcp.reshape(EP // C, C)
  dst2 = dstp.reshape(EP // C, C)
  onehot = (batch[:, None] == jnp.arange(G, dtype=jnp.int32)[None, :]
            ).astype(jnp.float32)

  h = x
  out = None
  for i, (w1, b1, w2, b2) in enumerate(params):
    p = _sc_agg(h, src2, dst2)
    if i < 2:
      h = _mlp(h, p, w1, b1, w2, b2, relu_out=True)
    else:
      out = _mlp_pool(h, p, w1, b1, w2, b2, onehot)
  return out


# speedup vs baseline: 1.3073x; 1.0459x over previous
import functools

import jax
import jax.numpy as jnp
from jax import lax
from jax.experimental import pallas as pl
from jax.experimental.pallas import tpu as pltpu
from jax.experimental.pallas import tpu_sc as plsc

N = 10000
E = 320000
D = 128
H = 128
G = 16

NC = 2
NS = 16
NW = NC * NS
K0 = 112
K1 = 48
C = 128
IB = 16
NB = 2
EP = NS * (K0 + K1) * C
NP = 10240
BR = 2000


def _sc_agg(h, src2, dst2):
  mesh = plsc.VectorSubcoreMesh(core_axis_name="c", subcore_axis_name="s",
                                num_cores=NC)

  @functools.partial(
      pl.kernel,
      out_type=jax.ShapeDtypeStruct((NC, NP, D), jnp.float32),
      mesh=mesh,
      scratch_types=[
          pltpu.VMEM((IB, C), jnp.int32),
          pltpu.VMEM((IB, C), jnp.int32),
          pltpu.VMEM((NB, C, D), jnp.float32),
          pltpu.VMEM_SHARED((NP, D), jnp.float32),
          pltpu.SemaphoreType.DMA((NB,)),
          pltpu.SemaphoreType.DMA((NB,)),
          pltpu.SemaphoreType.DMA,
      ],
  )
  def k(h_hbm, src_hbm, dst_hbm, out_hbm, src_v, dst_v, rows_v, agg,
        gsem, ssem, xsem):
    c = lax.axis_index("c")
    s = lax.axis_index("s")
    myk = jnp.where(c == 0, K0, K1)
    chunk0 = jnp.where(c == 0, s * K0, NS * K0 + s * K1)

    def zrow(i, carry):
      def zlane(j, carry2):
        rows_v[0, i, pl.ds(j * 16, 16)] = jnp.zeros((16,), jnp.float32)
        return carry2
      return lax.fori_loop(0, D // 16, zlane, carry)
    lax.fori_loop(0, C, zrow, 0)
    nz = NP // NS // C
    for t in range(nz):
      pltpu.async_copy(rows_v.at[0],
                       agg.at[pl.ds(s * (NP // NS) + t * C, C)], xsem)
    for t in range(nz):
      pltpu.make_async_copy(rows_v.at[0], agg.at[pl.ds(t * C, C)],
                            xsem).wait()
    plsc.subcore_barrier()

    def g_start(j, b):
      pltpu.async_copy(h_hbm.at[src_v.at[j]], rows_v.at[b], gsem.at[b])

    def g_wait(j, b):
      pltpu.make_async_copy(h_hbm.at[src_v.at[j]], rows_v.at[b],
                            gsem.at[b]).wait()

    def s_wait(j, b):
      pltpu.make_async_copy(rows_v.at[b], agg.at[dst_v.at[j]],
                            ssem.at[b]).wait()

    def block(bi, carry):
      base = chunk0 + bi * IB
      pltpu.async_copy(src_hbm.at[pl.ds(base, IB)], src_v, xsem)
      pltpu.async_copy(dst_hbm.at[pl.ds(base, IB)], dst_v, xsem)
      pltpu.make_async_copy(src_hbm.at[pl.ds(0, IB)], src_v, xsem).wait()
      pltpu.make_async_copy(dst_hbm.at[pl.ds(0, IB)], dst_v, xsem).wait()
      g_start(0, 0)

      def step(i, carry2):
        for b in range(NB):
          jl = i * NB + b
          g_wait(jl, b)
          pltpu.async_copy(rows_v.at[b], agg.at[dst_v.at[jl]], ssem.at[b],
                           add=True)

          @pl.when(jl >= 1)
          def _():
            s_wait(jl - 1, (b - 1) % NB)

          @pl.when(jl + 1 < IB)
          def _():
            g_start(jl + 1, (b + 1) % NB)
        return carry2
      lax.fori_loop(0, IB // NB, step, 0)
      s_wait(IB - 1, (IB - 1) % NB)
      return carry
    lax.fori_loop(0, myk // IB, block, 0)
    plsc.subcore_barrier()

    pltpu.sync_copy(agg.at[pl.ds(s * (NP // NS), NP // NS)],
                    out_hbm.at[c, pl.ds(s * (NP // NS), NP // NS)])

  return k(h, src2, dst2)


def _mlp_compute(h_ref, p_ref, w1_ref, b1_ref, w2_ref, b2_ref):
  m = h_ref[...] + p_ref[0]
  for q in range(1, NC):
    m = m + p_ref[q]
  m = jnp.dot(m, w1_ref[...], preferred_element_type=jnp.float32) + b1_ref[...]
  m = jnp.maximum(m, 0.0)
  m = jnp.dot(m, w2_ref[...], preferred_element_type=jnp.float32) + b2_ref[...]
  return m


def _mlp(h, p, w1, b1, w2, b2, relu_out):
  def body(h_ref, p_ref, w1_ref, b1_ref, w2_ref, b2_ref, o_ref):
    m = _mlp_compute(h_ref, p_ref, w1_ref, b1_ref, w2_ref, b2_ref)
    if relu_out:
      m = jnp.maximum(m, 0.0)
    o_ref[...] = m

  return pl.pallas_call(
      body,
      grid=(N // BR,),
      in_specs=[
          pl.BlockSpec((BR, D), lambda i: (i, 0)),
          pl.BlockSpec((NC, BR, D), lambda i: (0, i, 0)),
          pl.BlockSpec((D, H), lambda i: (0, 0)),
          pl.BlockSpec((1, H), lambda i: (0, 0)),
          pl.BlockSpec((H, H), lambda i: (0, 0)),
          pl.BlockSpec((1, H), lambda i: (0, 0)),
      ],
      out_specs=pl.BlockSpec((BR, H), lambda i: (i, 0)),
      out_shape=jax.ShapeDtypeStruct((N, H), jnp.float32),
  )(h, p, w1, b1.reshape(1, H), w2, b2.reshape(1, H))


def _mlp_pool(h, p, w1, b1, w2, b2, onehot):
  def body(h_ref, p_ref, w1_ref, b1_ref, w2_ref, b2_ref, oh_ref, o_ref):
    i = pl.program_id(0)
    m = _mlp_compute(h_ref, p_ref, w1_ref, b1_ref, w2_ref, b2_ref)
    rows = []
    for g in range(G):
      col = oh_ref[:, pl.ds(g, 1)]
      masked = jnp.where(col > 0.5, m, -jnp.inf)
      rows.append(jnp.max(masked, axis=0, keepdims=True))
    cur = jnp.concatenate(rows, axis=0)

    @pl.when(i == 0)
    def _():
      o_ref[...] = jnp.full((G, H), -jnp.inf, jnp.float32)
    o_ref[...] = jnp.maximum(o_ref[...], cur)

  return pl.pallas_call(
      body,
      grid=(N // BR,),
      in_specs=[
          pl.BlockSpec((BR, D), lambda i: (i, 0)),
          pl.BlockSpec((NC, BR, D), lambda i: (0, i, 0)),
          pl.BlockSpec((D, H), lambda i: (0, 0)),
          pl.BlockSpec((1, H), lambda i: (0, 0)),
          pl.BlockSpec((H, H), lambda i: (0, 0)),
          pl.BlockSpec((1, H), lambda i: (0, 0)),
          pl.BlockSpec((BR, G), lambda i: (i, 0)),
      ],
      out_specs=pl.BlockSpec((G, H), lambda i: (0, 0)),
      out_shape=jax.ShapeDtypeStruct((G, H), jnp.float32),
  )(h, p, w1, b1.reshape(1, H), w2, b2.reshape(1, H), onehot)


def kernel(x, edge_index, batch, W1_0, b1_0, W2_0, b2_0, W1_1, b1_1, W2_1,
           b2_1, W1_2, b1_2, W2_2, b2_2):
  params = [(W1_0, b1_0, W2_0, b2_0), (W1_1, b1_1, W2_1, b2_1),
            (W1_2, b1_2, W2_2, b2_2)]
  src = edge_index[0]
  dst = edge_index[1]
  srcp = jnp.concatenate([src, jnp.zeros((EP - E,), jnp.int32)])
  dstp = jnp.concatenate([dst, jnp.full((EP - E,), N, jnp.int32)])
  src2 = srcp.reshape(EP // C, C)
  dst2 = dstp.reshape(EP // C, C)
  onehot = (batch[:, None] == jnp.arange(G, dtype=jnp.int32)[None, :]
            ).astype(jnp.float32)

  h = x
  out = None
  for i, (w1, b1, w2, b2) in enumerate(params):
    p = _sc_agg(h, src2, dst2)
    if i < 2:
      h = _mlp(h, p, w1, b1, w2, b2, relu_out=True)
    else:
      out = _mlp_pool(h, p, w1, b1, w2, b2, onehot)
  return out


# speedup vs baseline: 1.3542x; 1.0359x over previous
import functools

import jax
import jax.numpy as jnp
from jax import lax
from jax.experimental import pallas as pl
from jax.experimental.pallas import tpu as pltpu
from jax.experimental.pallas import tpu_sc as plsc

N = 10000
E = 320000
D = 128
H = 128
G = 16

NC = 2
NS = 16
NW = NC * NS
K0 = 128
K1 = 32
C = 128
IB = 16
NB = 2
EP = NS * (K0 + K1) * C
NP = 10240
BR = 2000


def _sc_agg(h, src2, dst2):
  mesh = plsc.VectorSubcoreMesh(core_axis_name="c", subcore_axis_name="s",
                                num_cores=NC)

  @functools.partial(
      pl.kernel,
      out_type=jax.ShapeDtypeStruct((NC, NP, D), jnp.float32),
      mesh=mesh,
      scratch_types=[
          pltpu.VMEM((IB, C), jnp.int32),
          pltpu.VMEM((IB, C), jnp.int32),
          pltpu.VMEM((NB, C, D), jnp.float32),
          pltpu.VMEM_SHARED((NP, D), jnp.float32),
          pltpu.SemaphoreType.DMA((NB,)),
          pltpu.SemaphoreType.DMA((NB,)),
          pltpu.SemaphoreType.DMA,
      ],
  )
  def k(h_hbm, src_hbm, dst_hbm, out_hbm, src_v, dst_v, rows_v, agg,
        gsem, ssem, xsem):
    c = lax.axis_index("c")
    s = lax.axis_index("s")
    myk = jnp.where(c == 0, K0, K1)
    chunk0 = jnp.where(c == 0, s * K0, NS * K0 + s * K1)

    def zrow(i, carry):
      def zlane(j, carry2):
        rows_v[0, i, pl.ds(j * 16, 16)] = jnp.zeros((16,), jnp.float32)
        return carry2
      return lax.fori_loop(0, D // 16, zlane, carry)
    lax.fori_loop(0, C, zrow, 0)
    nz = NP // NS // C
    for t in range(nz):
      pltpu.async_copy(rows_v.at[0],
                       agg.at[pl.ds(s * (NP // NS) + t * C, C)], xsem)
    for t in range(nz):
      pltpu.make_async_copy(rows_v.at[0], agg.at[pl.ds(t * C, C)],
                            xsem).wait()
    plsc.subcore_barrier()

    def g_start(j, b):
      pltpu.async_copy(h_hbm.at[src_v.at[j]], rows_v.at[b], gsem.at[b])

    def g_wait(j, b):
      pltpu.make_async_copy(h_hbm.at[src_v.at[j]], rows_v.at[b],
                            gsem.at[b]).wait()

    def s_wait(j, b):
      pltpu.make_async_copy(rows_v.at[b], agg.at[dst_v.at[j]],
                            ssem.at[b]).wait()

    def block(bi, carry):
      base = chunk0 + bi * IB
      pltpu.async_copy(src_hbm.at[pl.ds(base, IB)], src_v, xsem)
      pltpu.async_copy(dst_hbm.at[pl.ds(base, IB)], dst_v, xsem)
      pltpu.make_async_copy(src_hbm.at[pl.ds(0, IB)], src_v, xsem).wait()
      pltpu.make_async_copy(dst_hbm.at[pl.ds(0, IB)], dst_v, xsem).wait()
      g_start(0, 0)

      def step(i, carry2):
        for b in range(NB):
          jl = i * NB + b
          g_wait(jl, b)
          pltpu.async_copy(rows_v.at[b], agg.at[dst_v.at[jl]], ssem.at[b],
                           add=True)

          @pl.when(jl >= 1)
          def _():
            s_wait(jl - 1, (b - 1) % NB)

          @pl.when(jl + 1 < IB)
          def _():
            g_start(jl + 1, (b + 1) % NB)
        return carry2
      lax.fori_loop(0, IB // NB, step, 0)
      s_wait(IB - 1, (IB - 1) % NB)
      return carry
    lax.fori_loop(0, myk // IB, block, 0)
    plsc.subcore_barrier()

    pltpu.sync_copy(agg.at[pl.ds(s * (NP // NS), NP // NS)],
                    out_hbm.at[c, pl.ds(s * (NP // NS), NP // NS)])

  return k(h, src2, dst2)


def _mlp_compute(h_ref, p_ref, w1_ref, b1_ref, w2_ref, b2_ref):
  m = h_ref[...] + p_ref[0]
  for q in range(1, NC):
    m = m + p_ref[q]
  m = jnp.dot(m, w1_ref[...], preferred_element_type=jnp.float32) + b1_ref[...]
  m = jnp.maximum(m, 0.0)
  m = jnp.dot(m, w2_ref[...], preferred_element_type=jnp.float32) + b2_ref[...]
  return m


def _mlp(h, p, w1, b1, w2, b2, relu_out):
  def body(h_ref, p_ref, w1_ref, b1_ref, w2_ref, b2_ref, o_ref):
    m = _mlp_compute(h_ref, p_ref, w1_ref, b1_ref, w2_ref, b2_ref)
    if relu_out:
      m = jnp.maximum(m, 0.0)
    o_ref[...] = m

  return pl.pallas_call(
      body,
      grid=(N // BR,),
      in_specs=[
          pl.BlockSpec((BR, D), lambda i: (i, 0)),
          pl.BlockSpec((NC, BR, D), lambda i: (0, i, 0)),
          pl.BlockSpec((D, H), lambda i: (0, 0)),
          pl.BlockSpec((1, H), lambda i: (0, 0)),
          pl.BlockSpec((H, H), lambda i: (0, 0)),
          pl.BlockSpec((1, H), lambda i: (0, 0)),
      ],
      out_specs=pl.BlockSpec((BR, H), lambda i: (i, 0)),
      out_shape=jax.ShapeDtypeStruct((N, H), jnp.float32),
  )(h, p, w1, b1.reshape(1, H), w2, b2.reshape(1, H))


def _mlp_pool(h, p, w1, b1, w2, b2, onehot):
  def body(h_ref, p_ref, w1_ref, b1_ref, w2_ref, b2_ref, oh_ref, o_ref):
    i = pl.program_id(0)
    m = _mlp_compute(h_ref, p_ref, w1_ref, b1_ref, w2_ref, b2_ref)
    rows = []
    for g in range(G):
      col = oh_ref[:, pl.ds(g, 1)]
      masked = jnp.where(col > 0.5, m, -jnp.inf)
      rows.append(jnp.max(masked, axis=0, keepdims=True))
    cur = jnp.concatenate(rows, axis=0)

    @pl.when(i == 0)
    def _():
      o_ref[...] = jnp.full((G, H), -jnp.inf, jnp.float32)
    o_ref[...] = jnp.maximum(o_ref[...], cur)

  return pl.pallas_call(
      body,
      grid=(N // BR,),
      in_specs=[
          pl.BlockSpec((BR, D), lambda i: (i, 0)),
          pl.BlockSpec((NC, BR, D), lambda i: (0, i, 0)),
          pl.BlockSpec((D, H), lambda i: (0, 0)),
          pl.BlockSpec((1, H), lambda i: (0, 0)),
          pl.BlockSpec((H, H), lambda i: (0, 0)),
          pl.BlockSpec((1, H), lambda i: (0, 0)),
          pl.BlockSpec((BR, G), lambda i: (i, 0)),
      ],
      out_specs=pl.BlockSpec((G, H), lambda i: (0, 0)),
      out_shape=jax.ShapeDtypeStruct((G, H), jnp.float32),
  )(h, p, w1, b1.reshape(1, H), w2, b2.reshape(1, H), onehot)


def kernel(x, edge_index, batch, W1_0, b1_0, W2_0, b2_0, W1_1, b1_1, W2_1,
           b2_1, W1_2, b1_2, W2_2, b2_2):
  params = [(W1_0, b1_0, W2_0, b2_0), (W1_1, b1_1, W2_1, b2_1),
            (W1_2, b1_2, W2_2, b2_2)]
  src = edge_index[0]
  dst = edge_index[1]
  srcp = jnp.concatenate([src, jnp.zeros((EP - E,), jnp.int32)])
  dstp = jnp.concatenate([dst, jnp.full((EP - E,), N, jnp.int32)])
  src2 = srcp.reshape(EP // C, C)
  dst2 = dstp.reshape(EP // C, C)
  onehot = (batch[:, None] == jnp.arange(G, dtype=jnp.int32)[None, :]
            ).astype(jnp.float32)

  h = x
  out = None
  for i, (w1, b1, w2, b2) in enumerate(params):
    p = _sc_agg(h, src2, dst2)
    if i < 2:
      h = _mlp(h, p, w1, b1, w2, b2, relu_out=True)
    else:
      out = _mlp_pool(h, p, w1, b1, w2, b2, onehot)
  return out


# speedup vs baseline: 1.5000x; 1.1077x over previous
import functools

import jax
import jax.numpy as jnp
from jax import lax
from jax.experimental import pallas as pl
from jax.experimental.pallas import tpu as pltpu
from jax.experimental.pallas import tpu_sc as plsc

N = 10000
E = 320000
D = 128
H = 128
G = 16

NC = 2
NS = 16
NW = NC * NS
K0 = 144
K1 = 16
C = 128
IB = 16
NB = 2
EP = NS * (K0 + K1) * C
NP = 10240
BR = 2000


def _sc_agg(h, src2, dst2):
  mesh = plsc.VectorSubcoreMesh(core_axis_name="c", subcore_axis_name="s",
                                num_cores=NC)

  @functools.partial(
      pl.kernel,
      out_type=jax.ShapeDtypeStruct((NC, NP, D), jnp.float32),
      mesh=mesh,
      scratch_types=[
          pltpu.VMEM((IB, C), jnp.int32),
          pltpu.VMEM((IB, C), jnp.int32),
          pltpu.VMEM((NB, C, D), jnp.float32),
          pltpu.VMEM_SHARED((NP, D), jnp.float32),
          pltpu.SemaphoreType.DMA((NB,)),
          pltpu.SemaphoreType.DMA((NB,)),
          pltpu.SemaphoreType.DMA,
      ],
  )
  def k(h_hbm, src_hbm, dst_hbm, out_hbm, src_v, dst_v, rows_v, agg,
        gsem, ssem, xsem):
    c = lax.axis_index("c")
    s = lax.axis_index("s")
    myk = jnp.where(c == 0, K0, K1)
    chunk0 = jnp.where(c == 0, s * K0, NS * K0 + s * K1)

    def zrow(i, carry):
      def zlane(j, carry2):
        rows_v[0, i, pl.ds(j * 16, 16)] = jnp.zeros((16,), jnp.float32)
        return carry2
      return lax.fori_loop(0, D // 16, zlane, carry)
    lax.fori_loop(0, C, zrow, 0)
    nz = NP // NS // C
    for t in range(nz):
      pltpu.async_copy(rows_v.at[0],
                       agg.at[pl.ds(s * (NP // NS) + t * C, C)], xsem)
    for t in range(nz):
      pltpu.make_async_copy(rows_v.at[0], agg.at[pl.ds(t * C, C)],
                            xsem).wait()
    plsc.subcore_barrier()

    def g_start(j, b):
      pltpu.async_copy(h_hbm.at[src_v.at[j]], rows_v.at[b], gsem.at[b])

    def g_wait(j, b):
      pltpu.make_async_copy(h_hbm.at[src_v.at[j]], rows_v.at[b],
                            gsem.at[b]).wait()

    def s_wait(j, b):
      pltpu.make_async_copy(rows_v.at[b], agg.at[dst_v.at[j]],
                            ssem.at[b]).wait()

    def block(bi, carry):
      base = chunk0 + bi * IB
      pltpu.async_copy(src_hbm.at[pl.ds(base, IB)], src_v, xsem)
      pltpu.async_copy(dst_hbm.at[pl.ds(base, IB)], dst_v, xsem)
      pltpu.make_async_copy(src_hbm.at[pl.ds(0, IB)], src_v, xsem).wait()
      pltpu.make_async_copy(dst_hbm.at[pl.ds(0, IB)], dst_v, xsem).wait()
      g_start(0, 0)

      def step(i, carry2):
        for b in range(NB):
          jl = i * NB + b
          g_wait(jl, b)
          pltpu.async_copy(rows_v.at[b], agg.at[dst_v.at[jl]], ssem.at[b],
                           add=True)

          @pl.when(jl >= 1)
          def _():
            s_wait(jl - 1, (b - 1) % NB)

          @pl.when(jl + 1 < IB)
          def _():
            g_start(jl + 1, (b + 1) % NB)
        return carry2
      lax.fori_loop(0, IB // NB, step, 0)
      s_wait(IB - 1, (IB - 1) % NB)
      return carry
    lax.fori_loop(0, myk // IB, block, 0)
    plsc.subcore_barrier()

    pltpu.sync_copy(agg.at[pl.ds(s * (NP // NS), NP // NS)],
                    out_hbm.at[c, pl.ds(s * (NP // NS), NP // NS)])

  return k(h, src2, dst2)


def _mlp_compute(h_ref, p_ref, w1_ref, b1_ref, w2_ref, b2_ref):
  m = h_ref[...] + p_ref[0]
  for q in range(1, NC):
    m = m + p_ref[q]
  m = jnp.dot(m, w1_ref[...], preferred_element_type=jnp.float32) + b1_ref[...]
  m = jnp.maximum(m, 0.0)
  m = jnp.dot(m, w2_ref[...], preferred_element_type=jnp.float32) + b2_ref[...]
  return m


def _mlp(h, p, w1, b1, w2, b2, relu_out):
  def body(h_ref, p_ref, w1_ref, b1_ref, w2_ref, b2_ref, o_ref):
    m = _mlp_compute(h_ref, p_ref, w1_ref, b1_ref, w2_ref, b2_ref)
    if relu_out:
      m = jnp.maximum(m, 0.0)
    o_ref[...] = m

  return pl.pallas_call(
      body,
      grid=(N // BR,),
      in_specs=[
          pl.BlockSpec((BR, D), lambda i: (i, 0)),
          pl.BlockSpec((NC, BR, D), lambda i: (0, i, 0)),
          pl.BlockSpec((D, H), lambda i: (0, 0)),
          pl.BlockSpec((1, H), lambda i: (0, 0)),
          pl.BlockSpec((H, H), lambda i: (0, 0)),
          pl.BlockSpec((1, H), lambda i: (0, 0)),
      ],
      out_specs=pl.BlockSpec((BR, H), lambda i: (i, 0)),
      out_shape=jax.ShapeDtypeStruct((N, H), jnp.float32),
  )(h, p, w1, b1.reshape(1, H), w2, b2.reshape(1, H))


def _mlp_pool(h, p, w1, b1, w2, b2, onehot):
  def body(h_ref, p_ref, w1_ref, b1_ref, w2_ref, b2_ref, oh_ref, o_ref):
    i = pl.program_id(0)
    m = _mlp_compute(h_ref, p_ref, w1_ref, b1_ref, w2_ref, b2_ref)
    rows = []
    for g in range(G):
      col = oh_ref[:, pl.ds(g, 1)]
      masked = jnp.where(col > 0.5, m, -jnp.inf)
      rows.append(jnp.max(masked, axis=0, keepdims=True))
    cur = jnp.concatenate(rows, axis=0)

    @pl.when(i == 0)
    def _():
      o_ref[...] = jnp.full((G, H), -jnp.inf, jnp.float32)
    o_ref[...] = jnp.maximum(o_ref[...], cur)

  return pl.pallas_call(
      body,
      grid=(N // BR,),
      in_specs=[
          pl.BlockSpec((BR, D), lambda i: (i, 0)),
          pl.BlockSpec((NC, BR, D), lambda i: (0, i, 0)),
          pl.BlockSpec((D, H), lambda i: (0, 0)),
          pl.BlockSpec((1, H), lambda i: (0, 0)),
          pl.BlockSpec((H, H), lambda i: (0, 0)),
          pl.BlockSpec((1, H), lambda i: (0, 0)),
          pl.BlockSpec((BR, G), lambda i: (i, 0)),
      ],
      out_specs=pl.BlockSpec((G, H), lambda i: (0, 0)),
      out_shape=jax.ShapeDtypeStruct((G, H), jnp.float32),
  )(h, p, w1, b1.reshape(1, H), w2, b2.reshape(1, H), onehot)


def kernel(x, edge_index, batch, W1_0, b1_0, W2_0, b2_0, W1_1, b1_1, W2_1,
           b2_1, W1_2, b1_2, W2_2, b2_2):
  params = [(W1_0, b1_0, W2_0, b2_0), (W1_1, b1_1, W2_1, b2_1),
            (W1_2, b1_2, W2_2, b2_2)]
  src = edge_index[0]
  dst = edge_index[1]
  srcp = jnp.concatenate([src, jnp.zeros((EP - E,), jnp.int32)])
  dstp = jnp.concatenate([dst, jnp.full((EP - E,), N, jnp.int32)])
  src2 = srcp.reshape(EP // C, C)
  dst2 = dstp.reshape(EP // C, C)
  onehot = (batch[:, None] == jnp.arange(G, dtype=jnp.int32)[None, :]
            ).astype(jnp.float32)

  h = x
  out = None
  for i, (w1, b1, w2, b2) in enumerate(params):
    p = _sc_agg(h, src2, dst2)
    if i < 2:
      h = _mlp(h, p, w1, b1, w2, b2, relu_out=True)
    else:
      out = _mlp_pool(h, p, w1, b1, w2, b2, onehot)
  return out
